# Initial kernel scaffold; baseline (speedup 1.0000x reference)
import functools

import jax
import jax.numpy as jnp
from jax import lax
from jax.experimental import pallas as pl
from jax.experimental.pallas import tpu as pltpu
from jax.experimental.pallas import tpu_sc as plsc

N_NODES = 10000
N_PAD = 10240
D = 128
E = 320000
NC, NS = 2, 16
NW = NC * NS
E_PER_W = E // NW
CHUNK = 125
NCHUNK = E_PER_W // CHUNK
RPT = N_PAD // NS
BLK = 1024
NBLK = N_PAD // BLK

_mesh = plsc.VectorSubcoreMesh(core_axis_name="c", subcore_axis_name="s")


def _degree_body(src_hbm, tgt_hbm, zrow_hbm, hist_hbm, maxp_hbm,
                 src_v, tgt_v, hist_v, max_v):
    c = lax.axis_index("c")
    s = lax.axis_index("s")
    wid = s * NC + c
    pltpu.sync_copy(src_hbm.at[wid], src_v)
    pltpu.sync_copy(tgt_hbm.at[wid], tgt_v)
    pltpu.sync_copy(zrow_hbm, hist_v)
    max_v[...] = jnp.zeros((16,), jnp.int32)
    ones = jnp.full((16,), 1.0, jnp.float32)

    def body(i, carry):
        s16 = src_v[pl.ds(i * 16, 16)]
        t16 = tgt_v[pl.ds(i * 16, 16)]
        plsc.addupdate_scatter(hist_v, [s16], ones)
        max_v[...] = jnp.maximum(max_v[...], jnp.maximum(s16, t16))
        return carry

    lax.fori_loop(0, E_PER_W // 16, body, 0)
    pltpu.sync_copy(hist_v, hist_hbm.at[wid])
    pltpu.sync_copy(max_v, maxp_hbm.at[wid])


@jax.jit
def _degree_call(src_a, tgt_a, zrow):
    return pl.kernel(
        _degree_body,
        out_type=(
            jax.ShapeDtypeStruct((NW, N_PAD), jnp.float32),
            jax.ShapeDtypeStruct((NW, 16), jnp.int32),
        ),
        mesh=_mesh,
        scratch_types=[
            pltpu.VMEM((E_PER_W,), jnp.int32),
            pltpu.VMEM((E_PER_W,), jnp.int32),
            pltpu.VMEM((N_PAD,), jnp.float32),
            pltpu.VMEM((16,), jnp.int32),
        ],
    )(src_a, tgt_a, zrow)


def _dis_block(hist_blk, nn, blk):
    cnt = jnp.sum(hist_blk, axis=0)
    row = lax.broadcasted_iota(jnp.int32, (BLK, 1), 0) + blk * BLK
    deg = cnt + (row < nn).astype(jnp.float32)
    return jnp.where(deg > 0.0, lax.rsqrt(deg), 0.0)


def _linear_body(nn_ref, x_ref, wt_ref, hist_ref, g_ref):
    blk = pl.program_id(0)
    dis = _dis_block(hist_ref[...], nn_ref[0], blk)
    h = jnp.dot(x_ref[...], wt_ref[...], preferred_element_type=jnp.float32)
    g_ref[...] = dis * h


@jax.jit
def _linear_call(nn, x_pad, wt, hists3):
    return pl.pallas_call(
        _linear_body,
        grid=(NBLK,),
        in_specs=[
            pl.BlockSpec(memory_space=pltpu.SMEM),
            pl.BlockSpec((BLK, D), lambda i: (i, 0)),
            pl.BlockSpec((D, D), lambda i: (0, 0)),
            pl.BlockSpec((NW, BLK, 1), lambda i: (0, i, 0)),
        ],
        out_specs=pl.BlockSpec((BLK, D), lambda i: (i, 0)),
        out_shape=jax.ShapeDtypeStruct((N_PAD, D), jnp.float32),
    )(nn, x_pad, wt, hists3)


def _scatter_body(g_hbm, src_hbm, tgt_hbm, zblk_hbm, part_hbm,
                  src_v, tgt_v, rows_a, rows_b, sem_a, sem_b, acc):
    c = lax.axis_index("c")
    s = lax.axis_index("s")
    wid = s * NC + c
    pltpu.sync_copy(src_hbm.at[wid], src_v)
    pltpu.sync_copy(tgt_hbm.at[wid], tgt_v)
    pltpu.sync_copy(zblk_hbm, acc.at[pl.ds(s * RPT, RPT)])
    plsc.subcore_barrier()

    pltpu.async_copy(g_hbm.at[src_v.at[0]], rows_a, sem_a)

    def body(jj, carry):
        j = jj * 2
        pltpu.make_async_copy(g_hbm.at[src_v.at[j]], rows_a, sem_a).wait()
        pltpu.async_copy(g_hbm.at[src_v.at[j + 1]], rows_b, sem_b)
        pltpu.sync_copy(rows_a, acc.at[tgt_v.at[j]], add=True)
        pltpu.make_async_copy(g_hbm.at[src_v.at[j + 1]], rows_b, sem_b).wait()

        @pl.when(j + 2 < NCHUNK)
        def _():
            pltpu.async_copy(g_hbm.at[src_v.at[j + 2]], rows_a, sem_a)

        pltpu.sync_copy(rows_b, acc.at[tgt_v.at[j + 1]], add=True)
        return carry

    lax.fori_loop(0, NCHUNK // 2, body, 0)
    plsc.subcore_barrier()
    pltpu.sync_copy(acc.at[pl.ds(s * RPT, RPT)],
                    part_hbm.at[pl.ds(c * N_PAD + s * RPT, RPT)])


@jax.jit
def _scatter_call(g, src_c, tgt_c, zblk):
    return pl.kernel(
        _scatter_body,
        out_type=jax.ShapeDtypeStruct((NC * N_PAD, D), jnp.float32),
        mesh=_mesh,
        scratch_types=[
            pltpu.VMEM((NCHUNK, CHUNK), jnp.int32),
            pltpu.VMEM((NCHUNK, CHUNK), jnp.int32),
            pltpu.VMEM((CHUNK, D), jnp.float32),
            pltpu.VMEM((CHUNK, D), jnp.float32),
            pltpu.SemaphoreType.DMA,
            pltpu.SemaphoreType.DMA,
            pltpu.VMEM_SHARED((N_PAD, D), jnp.float32),
        ],
    )(g, src_c, tgt_c, zblk)


def _out_body(nn_ref, p_ref, g_ref, hist_ref, o_ref):
    blk = pl.program_id(0)
    dis = _dis_block(hist_ref[...], nn_ref[0], blk)
    o_ref[...] = dis * (p_ref[0] + p_ref[1] + g_ref[...])


@jax.jit
def _out_call(nn, parts3, g, hists3):
    return pl.pallas_call(
        _out_body,
        grid=(NBLK,),
        in_specs=[
            pl.BlockSpec(memory_space=pltpu.SMEM),
            pl.BlockSpec((NC, BLK, D), lambda i: (0, i, 0)),
            pl.BlockSpec((BLK, D), lambda i: (i, 0)),
            pl.BlockSpec((NW, BLK, 1), lambda i: (0, i, 0)),
        ],
        out_specs=pl.BlockSpec((BLK, D), lambda i: (i, 0)),
        out_shape=jax.ShapeDtypeStruct((N_PAD, D), jnp.float32),
    )(nn, parts3, g, hists3)


def kernel(x, edge_index, W):
    src = edge_index[0].astype(jnp.int32)
    tgt = edge_index[1].astype(jnp.int32)
    x_pad = jnp.pad(x, ((0, N_PAD - N_NODES), (0, 0)))
    wt = W.T

    zrow = jnp.zeros((N_PAD,), jnp.float32)
    hists, maxp = _degree_call(src.reshape(NW, E_PER_W),
                               tgt.reshape(NW, E_PER_W), zrow)
    nn = (jnp.max(maxp) + 1).reshape(1).astype(jnp.int32)
    hists3 = hists.reshape(NW, N_PAD, 1)

    g = _linear_call(nn, x_pad, wt, hists3)

    zblk = jnp.zeros((RPT, D), jnp.float32)
    parts = _scatter_call(g, src.reshape(NW, NCHUNK, CHUNK),
                          tgt.reshape(NW, NCHUNK, CHUNK), zblk)

    out_pad = _out_call(nn, parts.reshape(NC, N_PAD, D), g, hists3)
    return out_pad[:N_NODES]


# speedup vs baseline: 16.9945x; 16.9945x over previous
import functools

import jax
import jax.numpy as jnp
from jax import lax
from jax.experimental import pallas as pl
from jax.experimental.pallas import tpu as pltpu
from jax.experimental.pallas import tpu_sc as plsc

N_NODES = 10000
N_PAD = 10240
D = 128
DH = D // 2
E = 320000
NC, NS = 2, 16
NW = NC * NS
E_PER_W = E // NW
E_PER_T = E // NS
CHUNK = 125
NCHUNK = E_PER_T // CHUNK
RPT = N_PAD // NS
BLK = 1024
NBLK = N_PAD // BLK

_mesh = plsc.VectorSubcoreMesh(core_axis_name="c", subcore_axis_name="s")


def _degree_body(src_hbm, tgt_hbm, zrow_hbm, hist_hbm, maxp_hbm,
                 src_v, tgt_v, hist_v, max_v):
    c = lax.axis_index("c")
    s = lax.axis_index("s")
    wid = s * NC + c
    pltpu.sync_copy(src_hbm.at[wid], src_v)
    pltpu.sync_copy(tgt_hbm.at[wid], tgt_v)
    pltpu.sync_copy(zrow_hbm, hist_v)
    max_v[...] = jnp.zeros((16,), jnp.int32)
    ones = jnp.full((16,), 1.0, jnp.float32)

    def body(i, carry):
        s16 = src_v[pl.ds(i * 16, 16)]
        t16 = tgt_v[pl.ds(i * 16, 16)]
        plsc.addupdate_scatter(hist_v, [s16], ones)
        max_v[...] = jnp.maximum(max_v[...], jnp.maximum(s16, t16))
        return carry

    lax.fori_loop(0, E_PER_W // 16, body, 0)
    pltpu.sync_copy(hist_v, hist_hbm.at[wid])
    pltpu.sync_copy(max_v, maxp_hbm.at[wid])


@jax.jit
def _degree_call(src_a, tgt_a, zrow):
    return pl.kernel(
        _degree_body,
        out_type=(
            jax.ShapeDtypeStruct((NW, N_PAD), jnp.float32),
            jax.ShapeDtypeStruct((NW, 16), jnp.int32),
        ),
        mesh=_mesh,
        scratch_types=[
            pltpu.VMEM((E_PER_W,), jnp.int32),
            pltpu.VMEM((E_PER_W,), jnp.int32),
            pltpu.VMEM((N_PAD,), jnp.float32),
            pltpu.VMEM((16,), jnp.int32),
        ],
        compiler_params=pltpu.CompilerParams(needs_layout_passes=False),
    )(src_a, tgt_a, zrow)


def _dis_block(hist_blk, nn, blk):
    cnt = jnp.sum(hist_blk, axis=0)
    row = lax.broadcasted_iota(jnp.int32, (BLK, 1), 0) + blk * BLK
    deg = cnt + (row < nn).astype(jnp.float32)
    return jnp.where(deg > 0.0, lax.rsqrt(deg), 0.0)


def _linear_body(nn_ref, x_ref, wt_ref, hist_ref, g_ref):
    blk = pl.program_id(0)
    dis = _dis_block(hist_ref[...], nn_ref[0], blk)
    h = jnp.dot(x_ref[...], wt_ref[...], preferred_element_type=jnp.float32)
    g = dis * h
    g_ref[0, :, :] = g[:, :DH]
    g_ref[1, :, :] = g[:, DH:]


@jax.jit
def _linear_call(nn, x_pad, wt, hists3):
    return pl.pallas_call(
        _linear_body,
        grid=(NBLK,),
        in_specs=[
            pl.BlockSpec(memory_space=pltpu.SMEM),
            pl.BlockSpec((BLK, D), lambda i: (i, 0)),
            pl.BlockSpec((D, D), lambda i: (0, 0)),
            pl.BlockSpec((NW, BLK, 1), lambda i: (0, i, 0)),
        ],
        out_specs=pl.BlockSpec((NC, BLK, DH), lambda i: (0, i, 0)),
        out_shape=jax.ShapeDtypeStruct((NC, N_PAD, DH), jnp.float32),
    )(nn, x_pad, wt, hists3)


def _scatter_body(g_hbm, src_hbm, tgt_hbm, zblk_hbm, part_hbm,
                  src_v, tgt_v, rows_a, rows_b, sem_a, sem_b, acc):
    c = lax.axis_index("c")
    s = lax.axis_index("s")
    g_half = g_hbm.at[c]
    pltpu.sync_copy(src_hbm.at[s], src_v)
    pltpu.sync_copy(tgt_hbm.at[s], tgt_v)
    pltpu.sync_copy(zblk_hbm, acc.at[pl.ds(s * RPT, RPT)])
    plsc.subcore_barrier()

    pltpu.async_copy(g_half.at[src_v.at[0]], rows_a, sem_a)

    def body(jj, carry):
        j = jj * 2
        pltpu.make_async_copy(g_half.at[src_v.at[j]], rows_a, sem_a).wait()
        pltpu.async_copy(g_half.at[src_v.at[j + 1]], rows_b, sem_b)
        pltpu.sync_copy(rows_a, acc.at[tgt_v.at[j]], add=True)
        pltpu.make_async_copy(g_half.at[src_v.at[j + 1]], rows_b, sem_b).wait()

        @pl.when(j + 2 < NCHUNK)
        def _():
            pltpu.async_copy(g_half.at[src_v.at[j + 2]], rows_a, sem_a)

        pltpu.sync_copy(rows_b, acc.at[tgt_v.at[j + 1]], add=True)
        return carry

    lax.fori_loop(0, NCHUNK // 2, body, 0)
    plsc.subcore_barrier()
    pltpu.sync_copy(acc.at[pl.ds(s * RPT, RPT)],
                    part_hbm.at[pl.ds(c * N_PAD + s * RPT, RPT)])


@jax.jit
def _scatter_call(g, src_c, tgt_c, zblk):
    return pl.kernel(
        _scatter_body,
        out_type=jax.ShapeDtypeStruct((NC * N_PAD, DH), jnp.float32),
        mesh=_mesh,
        scratch_types=[
            pltpu.VMEM((NCHUNK, CHUNK), jnp.int32),
            pltpu.VMEM((NCHUNK, CHUNK), jnp.int32),
            pltpu.VMEM((CHUNK, DH), jnp.float32),
            pltpu.VMEM((CHUNK, DH), jnp.float32),
            pltpu.SemaphoreType.DMA,
            pltpu.SemaphoreType.DMA,
            pltpu.VMEM_SHARED((N_PAD, DH), jnp.float32),
        ],
        compiler_params=pltpu.CompilerParams(use_tc_tiling_on_sc=False),
    )(g, src_c, tgt_c, zblk)


def _out_body(nn_ref, p_ref, g_ref, hist_ref, o_ref):
    blk = pl.program_id(0)
    dis = _dis_block(hist_ref[...], nn_ref[0], blk)
    acc = jnp.concatenate([p_ref[0], p_ref[1]], axis=1)
    g = jnp.concatenate([g_ref[0], g_ref[1]], axis=1)
    o_ref[...] = dis * (acc + g)


@jax.jit
def _out_call(nn, parts3, g, hists3):
    return pl.pallas_call(
        _out_body,
        grid=(NBLK,),
        in_specs=[
            pl.BlockSpec(memory_space=pltpu.SMEM),
            pl.BlockSpec((NC, BLK, DH), lambda i: (0, i, 0)),
            pl.BlockSpec((NC, BLK, DH), lambda i: (0, i, 0)),
            pl.BlockSpec((NW, BLK, 1), lambda i: (0, i, 0)),
        ],
        out_specs=pl.BlockSpec((BLK, D), lambda i: (i, 0)),
        out_shape=jax.ShapeDtypeStruct((N_PAD, D), jnp.float32),
    )(nn, parts3, g, hists3)


def kernel(x, edge_index, W):
    src = edge_index[0].astype(jnp.int32)
    tgt = edge_index[1].astype(jnp.int32)
    x_pad = jnp.pad(x, ((0, N_PAD - N_NODES), (0, 0)))
    wt = W.T

    zrow = jnp.zeros((N_PAD,), jnp.float32)
    hists, maxp = _degree_call(src.reshape(NW, E_PER_W),
                               tgt.reshape(NW, E_PER_W), zrow)
    nn = (jnp.max(maxp) + 1).reshape(1).astype(jnp.int32)
    hists3 = hists.reshape(NW, N_PAD, 1)

    g = _linear_call(nn, x_pad, wt, hists3)

    zblk = jnp.zeros((RPT, DH), jnp.float32)
    parts = _scatter_call(g, src.reshape(NS, NCHUNK, CHUNK),
                          tgt.reshape(NS, NCHUNK, CHUNK), zblk)

    out_pad = _out_call(nn, parts.reshape(NC, N_PAD, DH), g, hists3)
    return out_pad[:N_NODES]


# speedup vs baseline: 27.8381x; 1.6381x over previous
import functools

import jax
import jax.numpy as jnp
from jax import lax
from jax.experimental import pallas as pl
from jax.experimental.pallas import tpu as pltpu
from jax.experimental.pallas import tpu_sc as plsc

N_NODES = 10000
N_PAD = 10240
D = 128
DH = D // 2
E = 320000
NC, NS = 2, 16
NW = NC * NS
E_PER_W = E // NW
E_PER_T = E // NS
CHUNK = 125
NCHUNK = E_PER_T // CHUNK
RPT = N_PAD // NS
BLK = 1024
NBLK = N_PAD // BLK

_mesh = plsc.VectorSubcoreMesh(core_axis_name="c", subcore_axis_name="s")


def _degree_body(edge_hbm, zrow_hbm, hist_hbm, maxp_hbm,
                 src_v, tgt_v, hist_v, max_v):
    c = lax.axis_index("c")
    s = lax.axis_index("s")
    wid = s * NC + c
    pltpu.sync_copy(edge_hbm.at[0, wid], src_v)
    pltpu.sync_copy(edge_hbm.at[1, wid], tgt_v)
    pltpu.sync_copy(zrow_hbm, hist_v)
    max_v[...] = jnp.zeros((16,), jnp.int32)
    ones = jnp.full((16,), 1.0, jnp.float32)

    def body(i, carry):
        s16 = src_v[pl.ds(i * 16, 16)]
        t16 = tgt_v[pl.ds(i * 16, 16)]
        plsc.addupdate_scatter(hist_v, [s16], ones)
        max_v[...] = jnp.maximum(max_v[...], jnp.maximum(s16, t16))
        return carry

    lax.fori_loop(0, E_PER_W // 16, body, 0)
    pltpu.sync_copy(hist_v, hist_hbm.at[wid])
    pltpu.sync_copy(max_v, maxp_hbm.at[wid])


@jax.jit
def _degree_call(edges_a, zrow):
    return pl.kernel(
        _degree_body,
        out_type=(
            jax.ShapeDtypeStruct((NW, N_PAD), jnp.float32),
            jax.ShapeDtypeStruct((NW, 16), jnp.int32),
        ),
        mesh=_mesh,
        scratch_types=[
            pltpu.VMEM((E_PER_W,), jnp.int32),
            pltpu.VMEM((E_PER_W,), jnp.int32),
            pltpu.VMEM((N_PAD,), jnp.float32),
            pltpu.VMEM((16,), jnp.int32),
        ],
        compiler_params=pltpu.CompilerParams(needs_layout_passes=False),
    )(edges_a, zrow)


def _dis_body(nn_ref, hist_ref, dis_ref):
    cnt = jnp.sum(hist_ref[...], axis=0)
    r = lax.broadcasted_iota(jnp.int32, (N_PAD // 128, 128), 0)
    l = lax.broadcasted_iota(jnp.int32, (N_PAD // 128, 128), 1)
    node = r * 128 + l
    deg = cnt + (node < nn_ref[0]).astype(jnp.float32)
    dis_ref[...] = jnp.where(deg > 0.0, lax.rsqrt(deg), 0.0)


@jax.jit
def _dis_call(nn, hists4):
    return pl.pallas_call(
        _dis_body,
        in_specs=[
            pl.BlockSpec(memory_space=pltpu.SMEM),
            pl.BlockSpec((NW, N_PAD // 128, 128), lambda: (0, 0, 0)),
        ],
        out_specs=pl.BlockSpec((N_PAD // 128, 128), lambda: (0, 0)),
        out_shape=jax.ShapeDtypeStruct((N_PAD // 128, 128), jnp.float32),
    )(nn, hists4)


def _linear_body(x_ref, wt_ref, dis_ref, g_ref):
    h = jnp.dot(x_ref[...], wt_ref[...], preferred_element_type=jnp.float32)
    g = dis_ref[...] * h
    g_ref[0, :, :] = g[:, :DH]
    g_ref[1, :, :] = g[:, DH:]


@jax.jit
def _linear_call(x_pad, wt, dis_col):
    return pl.pallas_call(
        _linear_body,
        grid=(NBLK,),
        in_specs=[
            pl.BlockSpec((BLK, D), lambda i: (i, 0)),
            pl.BlockSpec((D, D), lambda i: (0, 0)),
            pl.BlockSpec((BLK, 1), lambda i: (i, 0)),
        ],
        out_specs=pl.BlockSpec((NC, BLK, DH), lambda i: (0, i, 0)),
        out_shape=jax.ShapeDtypeStruct((NC, N_PAD, DH), jnp.float32),
    )(x_pad, wt, dis_col)


def _scatter_body(g_hbm, edge_hbm, zblk_hbm, part_hbm,
                  src_v, tgt_v, rows_a, rows_b, sem_a, sem_b, acc):
    c = lax.axis_index("c")
    s = lax.axis_index("s")
    g_half = g_hbm.at[c]
    pltpu.sync_copy(edge_hbm.at[0, s], src_v)
    pltpu.sync_copy(edge_hbm.at[1, s], tgt_v)
    pltpu.sync_copy(zblk_hbm, acc.at[pl.ds(s * RPT, RPT)])
    plsc.subcore_barrier()

    pltpu.async_copy(g_half.at[src_v.at[0]], rows_a, sem_a)

    def body(jj, carry):
        j = jj * 2
        pltpu.make_async_copy(g_half.at[src_v.at[j]], rows_a, sem_a).wait()
        pltpu.async_copy(g_half.at[src_v.at[j + 1]], rows_b, sem_b)
        pltpu.sync_copy(rows_a, acc.at[tgt_v.at[j]], add=True)
        pltpu.make_async_copy(g_half.at[src_v.at[j + 1]], rows_b, sem_b).wait()

        @pl.when(j + 2 < NCHUNK)
        def _():
            pltpu.async_copy(g_half.at[src_v.at[j + 2]], rows_a, sem_a)

        pltpu.sync_copy(rows_b, acc.at[tgt_v.at[j + 1]], add=True)
        return carry

    lax.fori_loop(0, NCHUNK // 2, body, 0)
    plsc.subcore_barrier()
    pltpu.sync_copy(acc.at[pl.ds(s * RPT, RPT)],
                    part_hbm.at[pl.ds(c * N_PAD + s * RPT, RPT)])


@jax.jit
def _scatter_call(g, edges_c, zblk):
    return pl.kernel(
        _scatter_body,
        out_type=jax.ShapeDtypeStruct((NC * N_PAD, DH), jnp.float32),
        mesh=_mesh,
        scratch_types=[
            pltpu.VMEM((NCHUNK, CHUNK), jnp.int32),
            pltpu.VMEM((NCHUNK, CHUNK), jnp.int32),
            pltpu.VMEM((CHUNK, DH), jnp.float32),
            pltpu.VMEM((CHUNK, DH), jnp.float32),
            pltpu.SemaphoreType.DMA,
            pltpu.SemaphoreType.DMA,
            pltpu.VMEM_SHARED((N_PAD, DH), jnp.float32),
        ],
        compiler_params=pltpu.CompilerParams(use_tc_tiling_on_sc=False),
    )(g, edges_c, zblk)


def _out_body(p_ref, g_ref, dis_ref, o_ref):
    acc = jnp.concatenate([p_ref[0], p_ref[1]], axis=1)
    g = jnp.concatenate([g_ref[0], g_ref[1]], axis=1)
    o_ref[...] = dis_ref[...] * (acc + g)


@jax.jit
def _out_call(parts3, g, dis_col):
    return pl.pallas_call(
        _out_body,
        grid=(NBLK,),
        in_specs=[
            pl.BlockSpec((NC, BLK, DH), lambda i: (0, i, 0)),
            pl.BlockSpec((NC, BLK, DH), lambda i: (0, i, 0)),
            pl.BlockSpec((BLK, 1), lambda i: (i, 0)),
        ],
        out_specs=pl.BlockSpec((BLK, D), lambda i: (i, 0)),
        out_shape=jax.ShapeDtypeStruct((N_PAD, D), jnp.float32),
    )(parts3, g, dis_col)


def kernel(x, edge_index, W):
    edges = edge_index.astype(jnp.int32)
    x_pad = jnp.pad(x, ((0, N_PAD - N_NODES), (0, 0)))
    wt = W.T

    zrow = jnp.zeros((N_PAD,), jnp.float32)
    hists, maxp = _degree_call(edges.reshape(2, NW, E_PER_W), zrow)
    nn = (jnp.max(maxp) + 1).reshape(1).astype(jnp.int32)

    dis_flat = _dis_call(nn, hists.reshape(NW, N_PAD // 128, 128))
    dis_col = dis_flat.reshape(N_PAD, 1)

    g = _linear_call(x_pad, wt, dis_col)

    zblk = jnp.zeros((RPT, DH), jnp.float32)
    parts = _scatter_call(g, edges.reshape(2, NS, NCHUNK, CHUNK), zblk)

    out_pad = _out_call(parts.reshape(NC, N_PAD, DH), g, dis_col)
    return out_pad[:N_NODES]


# speedup vs baseline: 37.3803x; 1.3428x over previous
import functools

import jax
import jax.numpy as jnp
from jax import lax
from jax.experimental import pallas as pl
from jax.experimental.pallas import tpu as pltpu
from jax.experimental.pallas import tpu_sc as plsc

N = 10000
N_HIST = 10240
D = 128
DH = D // 2
E = 320000
NC, NS = 2, 16
NW = NC * NS
E_PER_W = E // NW
E_PER_T = E // NS
CHUNK = 125
NCHUNK = E_PER_T // CHUNK
NBUF = 4
RPT = N // NS
BLK = 1000
NBLK = N // BLK

_mesh = plsc.VectorSubcoreMesh(core_axis_name="c", subcore_axis_name="s")


def _degree_body(edge_hbm, zrow_hbm, hist_hbm, maxp_hbm,
                 src_v, tgt_v, hist_v, max_v):
    c = lax.axis_index("c")
    s = lax.axis_index("s")
    wid = s * NC + c
    pltpu.sync_copy(edge_hbm.at[0, pl.ds(wid * E_PER_W, E_PER_W)], src_v)
    pltpu.sync_copy(edge_hbm.at[1, pl.ds(wid * E_PER_W, E_PER_W)], tgt_v)
    pltpu.sync_copy(zrow_hbm, hist_v)
    max_v[...] = jnp.zeros((16,), jnp.int32)
    ones = jnp.full((16,), 1.0, jnp.float32)

    def body(i, carry):
        s16 = src_v[pl.ds(i * 16, 16)]
        t16 = tgt_v[pl.ds(i * 16, 16)]
        plsc.addupdate_scatter(hist_v, [s16], ones)
        max_v[...] = jnp.maximum(max_v[...], jnp.maximum(s16, t16))
        return carry

    lax.fori_loop(0, E_PER_W // 16, body, 0)
    pltpu.sync_copy(hist_v, hist_hbm.at[wid])
    pltpu.sync_copy(max_v, maxp_hbm.at[wid])


@jax.jit
def _degree_call(edges_f, zrow):
    return pl.kernel(
        _degree_body,
        out_type=(
            jax.ShapeDtypeStruct((NW, N_HIST), jnp.float32),
            jax.ShapeDtypeStruct((NW, 16), jnp.int32),
        ),
        mesh=_mesh,
        scratch_types=[
            pltpu.VMEM((E_PER_W,), jnp.int32),
            pltpu.VMEM((E_PER_W,), jnp.int32),
            pltpu.VMEM((N_HIST,), jnp.float32),
            pltpu.VMEM((16,), jnp.int32),
        ],
        compiler_params=pltpu.CompilerParams(
            needs_layout_passes=False, use_tc_tiling_on_sc=False),
    )(edges_f, zrow)


def _dis_body(nn_ref, hist_ref, dis_ref):
    cnt = jnp.sum(hist_ref[...], axis=0)
    r = lax.broadcasted_iota(jnp.int32, (N_HIST // 128, 128), 0)
    l = lax.broadcasted_iota(jnp.int32, (N_HIST // 128, 128), 1)
    node = r * 128 + l
    deg = cnt + (node < nn_ref[0]).astype(jnp.float32)
    dis_ref[...] = jnp.where(deg > 0.0, lax.rsqrt(deg), 0.0)


@jax.jit
def _dis_call(nn, hists4):
    return pl.pallas_call(
        _dis_body,
        in_specs=[
            pl.BlockSpec(memory_space=pltpu.SMEM),
            pl.BlockSpec((NW, N_HIST // 128, 128), lambda: (0, 0, 0)),
        ],
        out_specs=pl.BlockSpec((N_HIST // 128, 128), lambda: (0, 0)),
        out_shape=jax.ShapeDtypeStruct((N_HIST // 128, 128), jnp.float32),
    )(nn, hists4)


def _linear_body(x_ref, wt_ref, dis_ref, g_ref):
    h = jnp.dot(x_ref[...], wt_ref[...], preferred_element_type=jnp.float32)
    g = dis_ref[...] * h
    g_ref[0, :, :] = g[:, :DH]
    g_ref[1, :, :] = g[:, DH:]


@jax.jit
def _linear_call(x, wt, dis_col):
    return pl.pallas_call(
        _linear_body,
        grid=(NBLK,),
        in_specs=[
            pl.BlockSpec((BLK, D), lambda i: (i, 0)),
            pl.BlockSpec((D, D), lambda i: (0, 0)),
            pl.BlockSpec((BLK, 1), lambda i: (i, 0)),
        ],
        out_specs=pl.BlockSpec((NC, BLK, DH), lambda i: (0, i, 0)),
        out_shape=jax.ShapeDtypeStruct((NC, N, DH), jnp.float32),
    )(x, wt, dis_col)


def _scatter_body(g_hbm, edge_hbm, zblk_hbm, part_hbm,
                  src_v, tgt_v, rows, gsems, ssems, acc):
    c = lax.axis_index("c")
    s = lax.axis_index("s")
    g_half = g_hbm.at[c]
    pltpu.sync_copy(edge_hbm.at[0, s], src_v)
    pltpu.sync_copy(edge_hbm.at[1, s], tgt_v)
    pltpu.sync_copy(zblk_hbm, acc.at[pl.ds(s * RPT, RPT)])
    plsc.subcore_barrier()

    for k in range(NBUF):
        pltpu.async_copy(g_half.at[src_v.at[k]], rows[k], gsems[k])

    def body(jj, carry):
        j = jj * NBUF
        for k in range(NBUF):
            pltpu.make_async_copy(
                g_half.at[src_v.at[j + k]], rows[k], gsems[k]).wait()
            pltpu.async_copy(
                rows[k], acc.at[tgt_v.at[j + k]], ssems[k], add=True)
        for k in range(NBUF):
            pltpu.make_async_copy(
                rows[k], acc.at[tgt_v.at[j + k]], ssems[k]).wait()

            @pl.when(j + NBUF + k < NCHUNK)
            def _():
                pltpu.async_copy(
                    g_half.at[src_v.at[j + NBUF + k]], rows[k], gsems[k])
        return carry

    lax.fori_loop(0, NCHUNK // NBUF, body, 0)
    plsc.subcore_barrier()
    pltpu.sync_copy(acc.at[pl.ds(s * RPT, RPT)],
                    part_hbm.at[pl.ds(c * N + s * RPT, RPT)])


@jax.jit
def _scatter_call(g, edges_c, zblk):
    return pl.kernel(
        _scatter_body,
        out_type=jax.ShapeDtypeStruct((NC * N, DH), jnp.float32),
        mesh=_mesh,
        scratch_types=[
            pltpu.VMEM((NCHUNK, CHUNK), jnp.int32),
            pltpu.VMEM((NCHUNK, CHUNK), jnp.int32),
            [pltpu.VMEM((CHUNK, DH), jnp.float32) for _ in range(NBUF)],
            [pltpu.SemaphoreType.DMA for _ in range(NBUF)],
            [pltpu.SemaphoreType.DMA for _ in range(NBUF)],
            pltpu.VMEM_SHARED((N, DH), jnp.float32),
        ],
        compiler_params=pltpu.CompilerParams(use_tc_tiling_on_sc=False),
    )(g, edges_c, zblk)


def _out_body(p_ref, g_ref, dis_ref, o_ref):
    acc = jnp.concatenate([p_ref[0], p_ref[1]], axis=1)
    g = jnp.concatenate([g_ref[0], g_ref[1]], axis=1)
    o_ref[...] = dis_ref[...] * (acc + g)


@jax.jit
def _out_call(parts3, g, dis_col):
    return pl.pallas_call(
        _out_body,
        grid=(NBLK,),
        in_specs=[
            pl.BlockSpec((NC, BLK, DH), lambda i: (0, i, 0)),
            pl.BlockSpec((NC, BLK, DH), lambda i: (0, i, 0)),
            pl.BlockSpec((BLK, 1), lambda i: (i, 0)),
        ],
        out_specs=pl.BlockSpec((BLK, D), lambda i: (i, 0)),
        out_shape=jax.ShapeDtypeStruct((N, D), jnp.float32),
    )(parts3, g, dis_col)


def kernel(x, edge_index, W):
    edges = edge_index.astype(jnp.int32)
    wt = W.T

    zrow = jnp.zeros((N_HIST,), jnp.float32)
    hists, maxp = _degree_call(edges, zrow)
    nn = (jnp.max(maxp) + 1).reshape(1).astype(jnp.int32)

    dis_flat = _dis_call(nn, hists.reshape(NW, N_HIST // 128, 128))
    dis_col = dis_flat.reshape(N_HIST, 1)

    g = _linear_call(x, wt, dis_col)

    zblk = jnp.zeros((RPT, DH), jnp.float32)
    parts = _scatter_call(g, edges.reshape(2, NS, NCHUNK, CHUNK), zblk)

    return _out_call(parts.reshape(NC, N, DH), g, dis_col)


# speedup vs baseline: 38.1951x; 1.0218x over previous
import functools

import jax
import jax.numpy as jnp
from jax import lax
from jax.experimental import pallas as pl
from jax.experimental.pallas import tpu as pltpu
from jax.experimental.pallas import tpu_sc as plsc

N = 10000
N_HIST = 10240
D = 128
DH = D // 2
E = 320000
NC, NS = 2, 16
NW = NC * NS
E_PER_W = E // NW
E_PER_T = E // NS
CHUNK = 125
NCHUNK = E_PER_T // CHUNK
NBUF = 5
RPT = N // NS
BLK = 2000
NBLK = N // BLK

_mesh = plsc.VectorSubcoreMesh(core_axis_name="c", subcore_axis_name="s")


def _degree_body(edge_hbm, zrow_hbm, hist_hbm, maxp_hbm,
                 src_v, tgt_v, hist_v, max_v):
    c = lax.axis_index("c")
    s = lax.axis_index("s")
    wid = s * NC + c
    pltpu.sync_copy(edge_hbm.at[0, pl.ds(wid * E_PER_W, E_PER_W)], src_v)
    pltpu.sync_copy(edge_hbm.at[1, pl.ds(wid * E_PER_W, E_PER_W)], tgt_v)
    pltpu.sync_copy(zrow_hbm, hist_v)
    max_v[...] = jnp.zeros((16,), jnp.int32)
    ones = jnp.full((16,), 1.0, jnp.float32)

    def body(i, carry):
        s16 = src_v[pl.ds(i * 16, 16)]
        t16 = tgt_v[pl.ds(i * 16, 16)]
        plsc.addupdate_scatter(hist_v, [s16], ones)
        max_v[...] = jnp.maximum(max_v[...], jnp.maximum(s16, t16))
        return carry

    lax.fori_loop(0, E_PER_W // 16, body, 0)
    pltpu.sync_copy(hist_v, hist_hbm.at[wid])
    pltpu.sync_copy(max_v, maxp_hbm.at[wid])


@jax.jit
def _degree_call(edges_f, zrow):
    return pl.kernel(
        _degree_body,
        out_type=(
            jax.ShapeDtypeStruct((NW, N_HIST), jnp.float32),
            jax.ShapeDtypeStruct((NW, 16), jnp.int32),
        ),
        mesh=_mesh,
        scratch_types=[
            pltpu.VMEM((E_PER_W,), jnp.int32),
            pltpu.VMEM((E_PER_W,), jnp.int32),
            pltpu.VMEM((N_HIST,), jnp.float32),
            pltpu.VMEM((16,), jnp.int32),
        ],
        compiler_params=pltpu.CompilerParams(
            needs_layout_passes=False, use_tc_tiling_on_sc=False),
    )(edges_f, zrow)


def _dis_body(nn_ref, hist_ref, dis_ref):
    cnt = jnp.sum(hist_ref[...], axis=0)
    r = lax.broadcasted_iota(jnp.int32, (N_HIST // 128, 128), 0)
    l = lax.broadcasted_iota(jnp.int32, (N_HIST // 128, 128), 1)
    node = r * 128 + l
    deg = cnt + (node < nn_ref[0]).astype(jnp.float32)
    dis_ref[...] = jnp.where(deg > 0.0, lax.rsqrt(deg), 0.0)


@jax.jit
def _dis_call(nn, hists4):
    return pl.pallas_call(
        _dis_body,
        in_specs=[
            pl.BlockSpec(memory_space=pltpu.SMEM),
            pl.BlockSpec((NW, N_HIST // 128, 128), lambda: (0, 0, 0)),
        ],
        out_specs=pl.BlockSpec((N_HIST // 128, 128), lambda: (0, 0)),
        out_shape=jax.ShapeDtypeStruct((N_HIST // 128, 128), jnp.float32),
    )(nn, hists4)


def _matmul_body(x_ref, wt_ref, h_ref):
    h_ref[...] = jnp.dot(x_ref[...], wt_ref[...],
                         preferred_element_type=jnp.float32)


@jax.jit
def _matmul_call(x, wt):
    return pl.pallas_call(
        _matmul_body,
        grid=(NBLK,),
        in_specs=[
            pl.BlockSpec((BLK, D), lambda i: (i, 0)),
            pl.BlockSpec((D, D), lambda i: (0, 0)),
        ],
        out_specs=pl.BlockSpec((BLK, D), lambda i: (i, 0)),
        out_shape=jax.ShapeDtypeStruct((N, D), jnp.float32),
    )(x, wt)


def _scale_body(h_ref, dis_ref, g_ref):
    g = dis_ref[...] * h_ref[...]
    g_ref[0, :, :] = g[:, :DH]
    g_ref[1, :, :] = g[:, DH:]


@jax.jit
def _scale_call(h, dis_col):
    return pl.pallas_call(
        _scale_body,
        grid=(NBLK,),
        in_specs=[
            pl.BlockSpec((BLK, D), lambda i: (i, 0)),
            pl.BlockSpec((BLK, 1), lambda i: (i, 0)),
        ],
        out_specs=pl.BlockSpec((NC, BLK, DH), lambda i: (0, i, 0)),
        out_shape=jax.ShapeDtypeStruct((NC, N, DH), jnp.float32),
    )(h, dis_col)


def _scatter_body(g_hbm, edge_hbm, zblk_hbm, part_hbm,
                  src_v, tgt_v, rows, gsems, ssems, acc):
    c = lax.axis_index("c")
    s = lax.axis_index("s")
    g_half = g_hbm.at[c]
    pltpu.sync_copy(edge_hbm.at[0, s], src_v)
    pltpu.sync_copy(edge_hbm.at[1, s], tgt_v)
    pltpu.sync_copy(zblk_hbm, acc.at[pl.ds(s * RPT, RPT)])
    plsc.subcore_barrier()

    for k in range(NBUF):
        pltpu.async_copy(g_half.at[src_v.at[k]], rows[k], gsems[k])

    def body(jj, carry):
        j = jj * NBUF
        for k in range(NBUF):
            pltpu.make_async_copy(
                g_half.at[src_v.at[j + k]], rows[k], gsems[k]).wait()
            pltpu.async_copy(
                rows[k], acc.at[tgt_v.at[j + k]], ssems[k], add=True)
        for k in range(NBUF):
            pltpu.make_async_copy(
                rows[k], acc.at[tgt_v.at[j + k]], ssems[k]).wait()

            @pl.when(j + NBUF + k < NCHUNK)
            def _():
                pltpu.async_copy(
                    g_half.at[src_v.at[j + NBUF + k]], rows[k], gsems[k])
        return carry

    lax.fori_loop(0, NCHUNK // NBUF, body, 0)
    plsc.subcore_barrier()
    pltpu.sync_copy(acc.at[pl.ds(s * RPT, RPT)],
                    part_hbm.at[pl.ds(c * N + s * RPT, RPT)])


@jax.jit
def _scatter_call(g, edges_c, zblk):
    return pl.kernel(
        _scatter_body,
        out_type=jax.ShapeDtypeStruct((NC * N, DH), jnp.float32),
        mesh=_mesh,
        scratch_types=[
            pltpu.VMEM((NCHUNK, CHUNK), jnp.int32),
            pltpu.VMEM((NCHUNK, CHUNK), jnp.int32),
            [pltpu.VMEM((CHUNK, DH), jnp.float32) for _ in range(NBUF)],
            [pltpu.SemaphoreType.DMA for _ in range(NBUF)],
            [pltpu.SemaphoreType.DMA for _ in range(NBUF)],
            pltpu.VMEM_SHARED((N, DH), jnp.float32),
        ],
        compiler_params=pltpu.CompilerParams(use_tc_tiling_on_sc=False),
    )(g, edges_c, zblk)


def _out_body(p_ref, g_ref, dis_ref, o_ref):
    acc = jnp.concatenate([p_ref[0], p_ref[1]], axis=1)
    g = jnp.concatenate([g_ref[0], g_ref[1]], axis=1)
    o_ref[...] = dis_ref[...] * (acc + g)


@jax.jit
def _out_call(parts3, g, dis_col):
    return pl.pallas_call(
        _out_body,
        grid=(NBLK,),
        in_specs=[
            pl.BlockSpec((NC, BLK, DH), lambda i: (0, i, 0)),
            pl.BlockSpec((NC, BLK, DH), lambda i: (0, i, 0)),
            pl.BlockSpec((BLK, 1), lambda i: (i, 0)),
        ],
        out_specs=pl.BlockSpec((BLK, D), lambda i: (i, 0)),
        out_shape=jax.ShapeDtypeStruct((N, D), jnp.float32),
    )(parts3, g, dis_col)


def kernel(x, edge_index, W):
    edges = edge_index.astype(jnp.int32)
    wt = W.T

    zrow = jnp.zeros((N_HIST,), jnp.float32)
    hists, maxp = _degree_call(edges, zrow)
    h = _matmul_call(x, wt)
    nn = (jnp.max(maxp) + 1).reshape(1).astype(jnp.int32)

    dis_flat = _dis_call(nn, hists.reshape(NW, N_HIST // 128, 128))
    dis_col = dis_flat.reshape(N_HIST, 1)

    g = _scale_call(h, dis_col)

    zblk = jnp.zeros((RPT, DH), jnp.float32)
    parts = _scatter_call(g, edges.reshape(2, NS, NCHUNK, CHUNK), zblk)

    return _out_call(parts.reshape(NC, N, DH), g, dis_col)
